# baseline (speedup 1.0000x reference)
import functools

import jax
import jax.numpy as jnp
from jax.experimental import pallas as pl


def _contract_last(a, b):
    return jax.lax.dot_general(
        a, b, (((1,), (1,)), ((), ())), preferred_element_type=jnp.float32
    )


def _moe_kernel(x_ref, wg_ref, wup_ref, wdown_ref, o_ref):
    x = x_ref[...]
    h = _contract_last(x, wup_ref[...])
    h = h * jax.nn.sigmoid(h)
    out = _contract_last(h.astype(jnp.bfloat16), wdown_ref[...])
    g = _contract_last(x, wg_ref[...])
    m = jnp.max(g, axis=1, keepdims=True)
    e = jnp.exp(g - m)
    z = jnp.sum(e, axis=1, keepdims=True)
    m1 = jnp.max(e, axis=1, keepdims=True)
    iota = jax.lax.broadcasted_iota(jnp.int32, g.shape, 1)
    first = jnp.min(
        jnp.where(e == m1, iota, g.shape[1]), axis=1, keepdims=True
    )
    e2 = jnp.where(iota == first, 0.0, e)
    m2 = jnp.max(e2, axis=1, keepdims=True)
    s = (m1 + m2) / z
    scale = s / (s + 1e-9)
    o_ref[...] = out * scale


@functools.partial(jax.jit, static_argnames=("tm",))
def _run(xf, wg, wup, wdown, tm):
    t, d = xf.shape
    ed = wup.shape[0]
    return pl.pallas_call(
        _moe_kernel,
        grid=(t // tm,),
        in_specs=[
            pl.BlockSpec((tm, d), lambda i: (i, 0)),
            pl.BlockSpec(wg.shape, lambda i: (0, 0)),
            pl.BlockSpec((ed, d), lambda i: (0, 0)),
            pl.BlockSpec((d, ed), lambda i: (0, 0)),
        ],
        out_specs=pl.BlockSpec((tm, d), lambda i: (i, 0)),
        out_shape=jax.ShapeDtypeStruct((t, d), jnp.float32),
    )(xf, wg, wup, wdown)


def kernel(x, W_g, W_up, W_down):
    b, s, d = x.shape
    xf = x.reshape(b * s, d).astype(jnp.bfloat16)
    out = _run(
        xf,
        W_g.astype(jnp.bfloat16),
        W_up.astype(jnp.bfloat16),
        W_down.astype(jnp.bfloat16),
        tm=512,
    )
    return out.reshape(b, s, d)


# speedup vs baseline: 1.1150x; 1.1150x over previous
import functools

import jax
import jax.numpy as jnp
from jax.experimental import pallas as pl


def _contract_last(a, b, out_dtype=jnp.float32):
    return jax.lax.dot_general(
        a, b, (((1,), (1,)), ((), ())), preferred_element_type=out_dtype
    )


def _moe_kernel(x_ref, wg_ref, wup_ref, wdown_ref, o_ref):
    x = x_ref[...].astype(jnp.bfloat16)
    h = _contract_last(x, wup_ref[...])
    hb = h.astype(jnp.bfloat16)
    hb = hb * jax.nn.sigmoid(hb)
    out = _contract_last(hb, wdown_ref[...])
    g = _contract_last(x, wg_ref[...])
    m = jnp.max(g, axis=1, keepdims=True)
    e = jnp.exp(g - m)
    z = jnp.sum(e, axis=1, keepdims=True)
    m1 = jnp.max(e, axis=1, keepdims=True)
    iota = jax.lax.broadcasted_iota(jnp.int32, g.shape, 1)
    first = jnp.min(
        jnp.where(e == m1, iota, g.shape[1]), axis=1, keepdims=True
    )
    e2 = jnp.where(iota == first, 0.0, e)
    m2 = jnp.max(e2, axis=1, keepdims=True)
    s = (m1 + m2) / z
    scale = s / (s + 1e-9)
    o_ref[...] = out * scale


@functools.partial(jax.jit, static_argnames=("tm",))
def _run(xf, wg, wup, wdown, tm):
    t, d = xf.shape
    ed = wup.shape[0]
    return pl.pallas_call(
        _moe_kernel,
        grid=(t // tm,),
        in_specs=[
            pl.BlockSpec((tm, d), lambda i: (i, 0)),
            pl.BlockSpec(wg.shape, lambda i: (0, 0)),
            pl.BlockSpec((ed, d), lambda i: (0, 0)),
            pl.BlockSpec((d, ed), lambda i: (0, 0)),
        ],
        out_specs=pl.BlockSpec((tm, d), lambda i: (i, 0)),
        out_shape=jax.ShapeDtypeStruct((t, d), jnp.float32),
    )(xf, wg, wup, wdown)


def kernel(x, W_g, W_up, W_down):
    b, s, d = x.shape
    xf = x.reshape(b * s, d)
    out = _run(
        xf,
        W_g.astype(jnp.bfloat16),
        W_up.astype(jnp.bfloat16),
        W_down.astype(jnp.bfloat16),
        tm=512,
    )
    return out.reshape(b, s, d)


# speedup vs baseline: 1.2122x; 1.0872x over previous
import functools

import jax
import jax.numpy as jnp
from jax.experimental import pallas as pl
from jax.experimental.pallas import tpu as pltpu

_P = 16


def _contract_last(a, b):
    return jax.lax.dot_general(
        a, b, (((1,), (1,)), ((), ())), preferred_element_type=jnp.float32
    )


def _moe_kernel(x_ref, wg_ref, wupf_ref, wdownf_ref, o_ref, wub, wdb):
    i = pl.program_id(0)
    cu = wupf_ref.shape[0]
    cd = wdownf_ref.shape[0]

    @pl.when(i < _P)
    def _cast_weights():
        wub[pl.ds(i * cu, cu), :] = wupf_ref[...].astype(jnp.bfloat16)
        wdb[pl.ds(i * cd, cd), :] = wdownf_ref[...].astype(jnp.bfloat16)

    @pl.when(i >= _P)
    def _compute():
        x = x_ref[...].astype(jnp.bfloat16)
        h = _contract_last(x, wub[...])
        hb = h.astype(jnp.bfloat16)
        hb = hb * jax.nn.sigmoid(hb)
        out = _contract_last(hb, wdb[...])
        g = _contract_last(x, wg_ref[...].astype(jnp.bfloat16))
        m = jnp.max(g, axis=1, keepdims=True)
        e = jnp.exp(g - m)
        z = jnp.sum(e, axis=1, keepdims=True)
        m1 = jnp.max(e, axis=1, keepdims=True)
        iota = jax.lax.broadcasted_iota(jnp.int32, g.shape, 1)
        first = jnp.min(
            jnp.where(e == m1, iota, g.shape[1]), axis=1, keepdims=True
        )
        e2 = jnp.where(iota == first, 0.0, e)
        m2 = jnp.max(e2, axis=1, keepdims=True)
        s = (m1 + m2) / z
        scale = s / (s + 1e-9)
        o_ref[...] = out * scale


@functools.partial(jax.jit, static_argnames=("tm",))
def _run(xf, wg, wup, wdown, tm):
    t, d = xf.shape
    ed = wup.shape[0]
    cu = ed // _P
    cd = d // _P
    nt = t // tm
    return pl.pallas_call(
        _moe_kernel,
        grid=(_P + nt,),
        in_specs=[
            pl.BlockSpec((tm, d), lambda i: (jnp.maximum(i - _P, 0), 0)),
            pl.BlockSpec(wg.shape, lambda i: (0, 0)),
            pl.BlockSpec((cu, d), lambda i: (jnp.minimum(i, _P - 1), 0)),
            pl.BlockSpec((cd, ed), lambda i: (jnp.minimum(i, _P - 1), 0)),
        ],
        out_specs=pl.BlockSpec((tm, d), lambda i: (jnp.maximum(i - _P, 0), 0)),
        out_shape=jax.ShapeDtypeStruct((t, d), jnp.float32),
        scratch_shapes=[
            pltpu.VMEM((ed, d), jnp.bfloat16),
            pltpu.VMEM((d, ed), jnp.bfloat16),
        ],
    )(xf, wg, wup, wdown)


def kernel(x, W_g, W_up, W_down):
    b, s, d = x.shape
    xf = x.reshape(b * s, d)
    out = _run(xf, W_g, W_up, W_down, tm=256)
    return out.reshape(b, s, d)
